# Initial kernel scaffold; baseline (speedup 1.0000x reference)
import functools

import jax
import jax.numpy as jnp
from jax import lax
from jax.experimental import pallas as pl
from jax.experimental.pallas import tpu as pltpu
from jax.experimental.pallas import tpu_sc as plsc

B, T, NC, DC = 1024, 50, 26, 16
V, D = 1000000, 64
P = B * T
NW = 32
P_W = P // NW
CH = 64
N_CH = P_W // CH
ROWS = CH * NC
DOUT = D + DC


def _emb_body(table, idx, cont, out, idx_v, rows_v, out_v, sem):
    wid = lax.axis_index("s") * 2 + lax.axis_index("c")
    pos_base = wid * P_W

    def chunk_body(ci, carry):
        pos0 = pos_base + ci * CH
        i0 = pos0 * NC
        pltpu.sync_copy(idx.at[pl.ds(i0, ROWS)], idx_v)
        gather = pltpu.async_copy(table.at[idx_v], rows_v, sem)
        pltpu.sync_copy(cont.at[pl.ds(pos0, CH)], out_v.at[:, pl.ds(D, DC)])
        gather.wait()

        def pos_body(p, c2):
            r0 = p * NC
            for v in range(D // 16):
                sl = pl.ds(v * 16, 16)
                acc = rows_v[r0, sl]
                for c in range(1, NC):
                    acc = acc + rows_v[r0 + c, sl]
            out_v[p, sl] = acc
            return c2

        lax.fori_loop(0, CH, pos_body, 0)
        pltpu.sync_copy(out_v, out.at[pl.ds(pos0, CH)])
        return carry

    lax.fori_loop(0, N_CH, chunk_body, 0)


@jax.jit
def _embed_concat(table, idx_flat, cont2d):
    mesh = plsc.VectorSubcoreMesh(core_axis_name="c", subcore_axis_name="s")
    kern = functools.partial(
        pl.kernel,
        mesh=mesh,
        out_type=jax.ShapeDtypeStruct((P, DOUT), jnp.float32),
        scratch_types=[
            pltpu.VMEM((ROWS,), jnp.int32),
            pltpu.VMEM((ROWS, D), jnp.float32),
            pltpu.VMEM((CH, DOUT), jnp.float32),
            pltpu.SemaphoreType.DMA,
        ],
    )(_emb_body)
    return kern(table, idx_flat, cont2d)


def kernel(ContTensor, CatTensor, LabelTensor, DoseTensor, TimeDiffTensor,
           VTensor, VancoElTensor, PtList, LengList, embed_weight):
    idx_flat = CatTensor.reshape(-1).astype(jnp.int32)
    cont2d = ContTensor.reshape(P, DC)
    out = _embed_concat(embed_weight, idx_flat, cont2d)
    outEmb = out.reshape(B, T, DOUT)
    return (outEmb, LabelTensor, LengList, DoseTensor, TimeDiffTensor,
            VTensor, VancoElTensor, PtList)


# speedup vs baseline: 1.8300x; 1.8300x over previous
import functools

import jax
import jax.numpy as jnp
from jax import lax
from jax.experimental import pallas as pl
from jax.experimental.pallas import tpu as pltpu
from jax.experimental.pallas import tpu_sc as plsc

B, T, NC, DC = 1024, 50, 26, 16
V, D = 1000000, 64
P = B * T
NW = 32
P_W = P // NW
CH = 64
N_CH = P_W // CH
ROWS = CH * NC
DOUT = D + DC


def _emb_body(table, idx, cont, out, idx_v, rows_v, out_v, sem):
    wid = lax.axis_index("s") * 2 + lax.axis_index("c")
    pos_base = wid * P_W

    def chunk_body(ci, carry):
        pos0 = pos_base + ci * CH
        i0 = pos0 * NC
        pltpu.sync_copy(idx.at[pl.ds(i0, ROWS)], idx_v)
        gather = pltpu.async_copy(table.at[idx_v], rows_v, sem)
        pltpu.sync_copy(cont.at[pl.ds(pos0, CH)], out_v.at[:, pl.ds(D, DC)])
        gather.wait()

        def pos_body(p, c2):
            r0 = p * NC
            for v in range(D // 16):
                sl = pl.ds(v * 16, 16)
                acc = rows_v[r0, sl]
                for c in range(1, NC):
                    acc = acc + rows_v[r0 + c, sl]
                out_v[p, sl] = acc
            return c2

        lax.fori_loop(0, CH, pos_body, 0)
        pltpu.sync_copy(out_v, out.at[pl.ds(pos0, CH)])
        return carry

    lax.fori_loop(0, N_CH, chunk_body, 0)


@jax.jit
def _embed_concat(table, idx_flat, cont2d):
    mesh = plsc.VectorSubcoreMesh(core_axis_name="c", subcore_axis_name="s")
    kern = functools.partial(
        pl.kernel,
        mesh=mesh,
        out_type=jax.ShapeDtypeStruct((P, DOUT), jnp.float32),
        scratch_types=[
            pltpu.VMEM((ROWS,), jnp.int32),
            pltpu.VMEM((ROWS, D), jnp.float32),
            pltpu.VMEM((CH, DOUT), jnp.float32),
            pltpu.SemaphoreType.DMA,
        ],
        compiler_params=pltpu.CompilerParams(use_tc_tiling_on_sc=False),
    )(_emb_body)
    return kern(table, idx_flat, cont2d)


def kernel(ContTensor, CatTensor, LabelTensor, DoseTensor, TimeDiffTensor,
           VTensor, VancoElTensor, PtList, LengList, embed_weight):
    idx_flat = CatTensor.reshape(-1).astype(jnp.int32)
    cont2d = ContTensor.reshape(P, DC)
    out = _embed_concat(embed_weight, idx_flat, cont2d)
    outEmb = out.reshape(B, T, DOUT)
    return (outEmb, LabelTensor, LengList, DoseTensor, TimeDiffTensor,
            VTensor, VancoElTensor, PtList)


# speedup vs baseline: 2.1988x; 1.2016x over previous
import functools

import jax
import jax.numpy as jnp
from jax import lax
from jax.experimental import pallas as pl
from jax.experimental.pallas import tpu as pltpu
from jax.experimental.pallas import tpu_sc as plsc

B, T, NC, DC = 1024, 50, 26, 16
V, D = 1000000, 64
P = B * T
NW = 32
P_W = P // NW
CH = 16
N_CH = P_W // CH
ROWS = CH * NC
DOUT = D + DC
IDX_W = P_W * NC


def _emb_body(table, idx, cont, out, idx_v, rows0, rows1, out0, out1,
              g0, g1, w0, w1):
    wid = lax.axis_index("s") * 2 + lax.axis_index("c")
    pos_base = wid * P_W
    pltpu.sync_copy(idx.at[pl.ds(pos_base * NC, IDX_W)], idx_v)

    rows_b = (rows0, rows1)
    out_b = (out0, out1)
    gsem = (g0, g1)
    wsem = (w0, w1)

    def start_gather(c, par):
        pltpu.async_copy(
            table.at[idx_v.at[pl.ds(c * ROWS, ROWS)]], rows_b[par], gsem[par])

    start_gather(0, 0)
    start_gather(1, 1)

    @pl.loop(0, N_CH // 2)
    def _(g2):
        for par in range(2):
            c = g2 * 2 + par
            pos0 = pos_base + c * CH
            rows_v = rows_b[par]
            out_v = out_b[par]

            @pl.when(c >= 2)
            def _():
                pltpu.make_async_copy(
                    out_v, out.at[pl.ds(pos0, CH)], wsem[par]).wait()

            pltpu.sync_copy(cont.at[pl.ds(pos0, CH)],
                            out_v.at[:, pl.ds(D, DC)])
            pltpu.make_async_copy(
                table.at[idx_v.at[pl.ds(c * ROWS, ROWS)]], rows_v,
                gsem[par]).wait()

            @plsc.parallel_loop(0, CH)
            def _(p):
                r0 = p * NC
                for v in range(D // 16):
                    sl = pl.ds(v * 16, 16)
                    acc = rows_v[r0, sl]
                    for cc in range(1, NC):
                        acc = acc + rows_v[r0 + cc, sl]
                    out_v[p, sl] = acc

            @pl.when(c + 2 < N_CH)
            def _():
                start_gather(c + 2, par)

            pltpu.async_copy(out_v, out.at[pl.ds(pos0, CH)], wsem[par])

    pltpu.make_async_copy(out0, out.at[pl.ds(pos_base, CH)], w0).wait()
    pltpu.make_async_copy(out1, out.at[pl.ds(pos_base, CH)], w1).wait()


@jax.jit
def _embed_concat(table, idx_flat, cont2d):
    mesh = plsc.VectorSubcoreMesh(core_axis_name="c", subcore_axis_name="s")
    kern = functools.partial(
        pl.kernel,
        mesh=mesh,
        out_type=jax.ShapeDtypeStruct((P, DOUT), jnp.float32),
        scratch_types=[
            pltpu.VMEM((IDX_W,), jnp.int32),
            pltpu.VMEM((ROWS, D), jnp.float32),
            pltpu.VMEM((ROWS, D), jnp.float32),
            pltpu.VMEM((CH, DOUT), jnp.float32),
            pltpu.VMEM((CH, DOUT), jnp.float32),
            pltpu.SemaphoreType.DMA,
            pltpu.SemaphoreType.DMA,
            pltpu.SemaphoreType.DMA,
            pltpu.SemaphoreType.DMA,
        ],
        compiler_params=pltpu.CompilerParams(use_tc_tiling_on_sc=False),
    )(_emb_body)
    return kern(table, idx_flat, cont2d)


def kernel(ContTensor, CatTensor, LabelTensor, DoseTensor, TimeDiffTensor,
           VTensor, VancoElTensor, PtList, LengList, embed_weight):
    idx_flat = CatTensor.reshape(-1).astype(jnp.int32)
    cont2d = ContTensor.reshape(P, DC)
    out = _embed_concat(embed_weight, idx_flat, cont2d)
    outEmb = out.reshape(B, T, DOUT)
    return (outEmb, LabelTensor, LengList, DoseTensor, TimeDiffTensor,
            VTensor, VancoElTensor, PtList)


# speedup vs baseline: 2.2078x; 1.0041x over previous
import functools

import jax
import jax.numpy as jnp
from jax import lax
from jax.experimental import pallas as pl
from jax.experimental.pallas import tpu as pltpu
from jax.experimental.pallas import tpu_sc as plsc

B, T, NC, DC = 1024, 50, 26, 16
V, D = 1000000, 64
P = B * T
NW = 32
P_W = P // NW
CH = 16
N_CH = P_W // CH
ROWS = CH * NC
DOUT = D + DC
IDX_W = P_W * NC


def _emb_body(table, cat2, cont, out, idx_v, rows0, rows1, out0, out1,
              g0, g1, w0, w1):
    wid = lax.axis_index("s") * 2 + lax.axis_index("c")
    pos_base = wid * P_W
    pltpu.sync_copy(cat2.at[pl.ds(wid * N_CH, N_CH)], idx_v)

    rows_b = (rows0, rows1)
    out_b = (out0, out1)
    gsem = (g0, g1)
    wsem = (w0, w1)

    def start_gather(c, par):
        pltpu.async_copy(
            table.at[idx_v.at[c]], rows_b[par], gsem[par])

    start_gather(0, 0)
    start_gather(1, 1)

    @pl.loop(0, N_CH // 2)
    def _(g2):
        for par in range(2):
            c = g2 * 2 + par
            pos0 = pos_base + c * CH
            rows_v = rows_b[par]
            out_v = out_b[par]

            @pl.when(c >= 2)
            def _():
                pltpu.make_async_copy(
                    out_v, out.at[pl.ds(pos0, CH)], wsem[par]).wait()

            pltpu.sync_copy(cont.at[pl.ds(pos0, CH)],
                            out_v.at[:, pl.ds(D, DC)])
            pltpu.make_async_copy(
                table.at[idx_v.at[c]], rows_v,
                gsem[par]).wait()

            @plsc.parallel_loop(0, CH)
            def _(p):
                r0 = p * NC
                for v in range(D // 16):
                    sl = pl.ds(v * 16, 16)
                    acc = rows_v[r0, sl]
                    for cc in range(1, NC):
                        acc = acc + rows_v[r0 + cc, sl]
                    out_v[p, sl] = acc

            @pl.when(c + 2 < N_CH)
            def _():
                start_gather(c + 2, par)

            pltpu.async_copy(out_v, out.at[pl.ds(pos0, CH)], wsem[par])

    pltpu.make_async_copy(out0, out.at[pl.ds(pos_base, CH)], w0).wait()
    pltpu.make_async_copy(out1, out.at[pl.ds(pos_base, CH)], w1).wait()


@jax.jit
def _embed_concat(table, cat2, cont2d):
    mesh = plsc.VectorSubcoreMesh(core_axis_name="c", subcore_axis_name="s")
    kern = functools.partial(
        pl.kernel,
        mesh=mesh,
        out_type=jax.ShapeDtypeStruct((P, DOUT), jnp.float32),
        scratch_types=[
            pltpu.VMEM((N_CH, ROWS), jnp.int32),
            pltpu.VMEM((ROWS, D), jnp.float32),
            pltpu.VMEM((ROWS, D), jnp.float32),
            pltpu.VMEM((CH, DOUT), jnp.float32),
            pltpu.VMEM((CH, DOUT), jnp.float32),
            pltpu.SemaphoreType.DMA,
            pltpu.SemaphoreType.DMA,
            pltpu.SemaphoreType.DMA,
            pltpu.SemaphoreType.DMA,
        ],
        compiler_params=pltpu.CompilerParams(use_tc_tiling_on_sc=False),
    )(_emb_body)
    return kern(table, cat2, cont2d)


def kernel(ContTensor, CatTensor, LabelTensor, DoseTensor, TimeDiffTensor,
           VTensor, VancoElTensor, PtList, LengList, embed_weight):
    cat2 = CatTensor.astype(jnp.int32).reshape(P // CH, ROWS)
    out = _embed_concat(embed_weight, cat2, ContTensor.reshape(P, DC))
    outEmb = out.reshape(B, T, DOUT)
    return (outEmb, LabelTensor, LengList, DoseTensor, TimeDiffTensor,
            VTensor, VancoElTensor, PtList)
